# baseline (speedup 1.0000x reference)
import functools

import jax
import jax.numpy as jnp
from jax import lax
from jax.experimental import pallas as pl
from jax.experimental.pallas import tpu as pltpu
from jax.experimental.pallas import tpu_sc as plsc

N_ROWS = 320000
D = 128
S = 512
NC, NS = 2, 16
NW = NC * NS
ROWS_PER_TILE = N_ROWS // NW
C = 125
NCHUNK = ROWS_PER_TILE // C
NPAIR = NCHUNK // 2
SEG_PER_TILE = S // NS
CW = 16
LANES = 16
NGROUP = C // LANES
TAIL = C - NGROUP * LANES


def _sc_partial_segsum(x, idxp, ones_cw, ident):
  mesh = plsc.VectorSubcoreMesh(
      core_axis_name="c", subcore_axis_name="s", num_cores=NC, num_subcores=NS)

  @functools.partial(
      pl.kernel,
      out_type=(
          jax.ShapeDtypeStruct((NC * S, D), jnp.float32),
          jax.ShapeDtypeStruct((NC * S, CW), jnp.float32),
      ),
      mesh=mesh,
      compiler_params=pltpu.CompilerParams(use_tc_tiling_on_sc=False,
                                           needs_layout_passes=False),
      scratch_types=[
          pltpu.VMEM((NCHUNK, 128), jnp.int32),
          [pltpu.VMEM((C, D), jnp.float32)] * 2,
          pltpu.VMEM((S, D), jnp.float32),
          pltpu.VMEM((128, CW), jnp.float32),
          pltpu.VMEM((SEG_PER_TILE, CW), jnp.float32),
          pltpu.VMEM((S // 128, 128), jnp.int32),
          pltpu.VMEM_SHARED((S, D), jnp.float32),
          pltpu.VMEM_SHARED((S, CW), jnp.float32),
          [pltpu.SemaphoreType.DMA] * 2,
          [pltpu.SemaphoreType.DMA] * 2,
          pltpu.SemaphoreType.DMA,
      ],
  )
  def k(x_hbm, idxp_hbm, ones_hbm, ident_hbm, psums_hbm, pcnts_hbm,
        idx_p, xbufs, acc, ones_v, zc, ident_v, sums_sh, cnts_sh,
        gsems, csems, msem):
    cid = lax.axis_index("c")
    sid = lax.axis_index("s")
    wid = cid * NS + sid
    row0 = wid * ROWS_PER_TILE

    pltpu.sync_copy(idxp_hbm.at[wid], idx_p)
    pltpu.sync_copy(ones_hbm.at[pl.ds(0, 128)], ones_v)
    pltpu.sync_copy(ones_hbm.at[pl.ds(128, SEG_PER_TILE)], zc)
    pltpu.sync_copy(ident_hbm, ident_v)

    zeros16 = jnp.zeros((LANES,), jnp.float32)

    def zacc(i, _):
      r = i // (D // LANES)
      acc[r, pl.ds((i % (D // LANES)) * LANES, LANES)] = zeros16
      return 0
    lax.fori_loop(0, S * (D // LANES), zacc, 0)

    pltpu.sync_copy(acc.at[pl.ds(0, SEG_PER_TILE)],
                    sums_sh.at[pl.ds(sid * SEG_PER_TILE, SEG_PER_TILE)])
    pltpu.sync_copy(zc, cnts_sh.at[pl.ds(sid * SEG_PER_TILE, SEG_PER_TILE)])
    plsc.subcore_barrier()

    def gstart(j, b):
      pltpu.async_copy(x_hbm.at[pl.ds(row0 + j * C, C)], xbufs[b], gsems[b])

    def gwait(b):
      pltpu.make_async_copy(x_hbm.at[pl.ds(0, C)], xbufs[b], gsems[b]).wait()

    lane_iota = lax.iota(jnp.int32, LANES)

    def reduce_chunk(j, b):
      xb = xbufs[b]

      def do_rows(r0, iv, nrows):
        for u in range(nrows):
          segv = jnp.full((LANES,), iv[u], jnp.int32)
          for c in range(D // LANES):
            v = xb[r0 + u, pl.ds(c * LANES, LANES)]
            plsc.addupdate_scatter(acc, [segv, lane_iota + (c * LANES)], v)

      def row_group(g, _):
        r0 = g * LANES
        iv = idx_p[j, pl.ds(r0, LANES)]
        do_rows(r0, iv, LANES)
        return 0
      lax.fori_loop(0, NGROUP, row_group, 0)
      iv_t = idx_p[j, pl.ds(NGROUP * LANES, LANES)]
      do_rows(NGROUP * LANES, iv_t, TAIL)

    gstart(0, 0)
    def pair(p, _):
      j0 = 2 * p
      gwait(0)
      gstart(j0 + 1, 1)
      d0 = pltpu.async_copy(ones_v, cnts_sh.at[idx_p.at[j0]], csems[0],
                            add=True)
      reduce_chunk(j0, 0)
      gwait(1)
      @pl.when(p < NPAIR - 1)
      def _():
        gstart(j0 + 2, 0)
      d1 = pltpu.async_copy(ones_v, cnts_sh.at[idx_p.at[j0 + 1]], csems[1],
                            add=True)
      reduce_chunk(j0 + 1, 1)
      d0.wait()
      d1.wait()
      return 0
    lax.fori_loop(0, NPAIR, pair, 0)

    for q in range(S // 128):
      pltpu.async_copy(acc.at[pl.ds(q * 128, 128)],
                       sums_sh.at[ident_v.at[q]],
                       msem, add=True).wait()
    plsc.subcore_barrier()

    pltpu.sync_copy(sums_sh.at[pl.ds(sid * SEG_PER_TILE, SEG_PER_TILE)],
                    acc.at[pl.ds(0, SEG_PER_TILE)])
    pltpu.sync_copy(acc.at[pl.ds(0, SEG_PER_TILE)],
                    psums_hbm.at[pl.ds(cid * S + sid * SEG_PER_TILE,
                                       SEG_PER_TILE)])
    pltpu.sync_copy(cnts_sh.at[pl.ds(sid * SEG_PER_TILE, SEG_PER_TILE)], zc)
    pltpu.sync_copy(zc, pcnts_hbm.at[pl.ds(cid * S + sid * SEG_PER_TILE,
                                           SEG_PER_TILE)])

  return k(x, idxp, ones_cw, ident)


def _combine(psums, pcnts):
  def body(ps_ref, pc_ref, o_ref):
    sums = ps_ref[0] + ps_ref[1]
    cnts = pc_ref[0, :, 0:1] + pc_ref[1, :, 0:1]
    o_ref[...] = sums / cnts
  return pl.pallas_call(
      body,
      out_shape=jax.ShapeDtypeStruct((S, D), jnp.float32),
  )(psums.reshape(NC, S, D), pcnts.reshape(NC, S, CW))


def kernel(x, batch_index):
  idx2d = batch_index.astype(jnp.int32).reshape(NW, NCHUNK, C)
  idxp = jnp.pad(idx2d, ((0, 0), (0, 0), (0, 128 - C)))
  ones_cw = jnp.concatenate([jnp.ones((C, CW), jnp.float32),
                             jnp.zeros((128 - C + SEG_PER_TILE, CW), jnp.float32)])
  ident = jnp.arange(S, dtype=jnp.int32).reshape(S // 128, 128)
  psums, pcnts = _sc_partial_segsum(x, idxp, ones_cw, ident)
  return _combine(psums, pcnts)
